# baseline (speedup 1.0000x reference)
import functools

import jax
import jax.numpy as jnp
from jax import lax
from jax.experimental import pallas as pl
from jax.experimental.pallas import tpu as pltpu
from jax.experimental.pallas import tpu_sc as plsc

_SC_MESH = dict(core_axis_name="c", subcore_axis_name="s")


def _sc_emb_gather(item_ids, user_ids, item_embT, user_embT):
    B = item_ids.shape[0]
    D, V = item_embT.shape
    info = plsc.get_sparse_core_info()
    NC, NS, L = info.num_cores, info.num_subcores, info.num_lanes
    NW = NC * NS
    bpw = B // NW
    RING = 4

    @functools.partial(
        pl.kernel,
        mesh=plsc.VectorSubcoreMesh(**_SC_MESH),
        out_type=(
            jax.ShapeDtypeStruct((B, D), jnp.float32),
            jax.ShapeDtypeStruct((B, D), jnp.float32),
        ),
        scratch_types=[
            pltpu.SMEM((bpw,), jnp.int32),
            pltpu.SMEM((bpw,), jnp.int32),
            pltpu.VMEM((bpw,), jnp.int32),
            pltpu.VMEM((bpw,), jnp.int32),
            pltpu.VMEM((RING, D, 128), jnp.float32),
            pltpu.VMEM((RING, D, 128), jnp.float32),
            pltpu.VMEM((bpw, D), jnp.float32),
            pltpu.VMEM((bpw, D), jnp.float32),
            [pltpu.SemaphoreType.DMA] * RING,
            [pltpu.SemaphoreType.DMA] * RING,
        ],
        compiler_params=pltpu.CompilerParams(
            use_tc_tiling_on_sc=True, needs_layout_passes=False),
    )
    def gather_kernel(iid_hbm, uid_hbm, iembT_hbm, uembT_hbm,
                      i_out, u_out,
                      iids_sm, uids_sm, iidx, uidx, islab, uslab,
                      irows, urows, isem, usem):
        wid = lax.axis_index("s") * NC + lax.axis_index("c")
        base = wid * bpw
        pltpu.sync_copy(iid_hbm.at[pl.ds(base, bpw)], iidx)
        pltpu.sync_copy(uid_hbm.at[pl.ds(base, bpw)], uidx)
        for j in range(bpw // L):
            iv = iidx[pl.ds(j * L, L)]
            uv = uidx[pl.ds(j * L, L)]
            for k in range(L):
                iids_sm[j * L + k] = iv[k]
                uids_sm[j * L + k] = uv[k]

        def issue(r, b):
            ic = pl.multiple_of(iids_sm[r] & ~127, 128)
            uc = pl.multiple_of(uids_sm[r] & ~127, 128)
            pltpu.async_copy(iembT_hbm.at[:, pl.ds(ic, 128)],
                             islab.at[b], isem[b])
            pltpu.async_copy(uembT_hbm.at[:, pl.ds(uc, 128)],
                             uslab.at[b], usem[b])

        for b in range(RING):
            issue(b, b)

        nG = bpw // RING
        rowv = lax.iota(jnp.int32, L)

        def body(g, _):
            for b in range(RING):
                r = g * RING + b
                pltpu.make_async_copy(
                    iembT_hbm.at[:, pl.ds(0, 128)], islab.at[b],
                    isem[b]).wait()
                pltpu.make_async_copy(
                    uembT_hbm.at[:, pl.ds(0, 128)], uslab.at[b],
                    usem[b]).wait()
                iid = iids_sm[r]
                uid = uids_sm[r]
                icol = jnp.broadcast_to(iid & 127, (L,))
                ucol = jnp.broadcast_to(uid & 127, (L,))
                for h in range(D // L):
                    irows[r, pl.ds(h * L, L)] = plsc.load_gather(
                        islab.at[b], [rowv + h * L, icol])
                    urows[r, pl.ds(h * L, L)] = plsc.load_gather(
                        uslab.at[b], [rowv + h * L, ucol])

                @pl.when(g < nG - 1)
                def _():
                    issue(r + RING, b)

            return 0

        lax.fori_loop(0, nG, body, 0)
        pltpu.sync_copy(irows, i_out.at[pl.ds(base, bpw)])
        pltpu.sync_copy(urows, u_out.at[pl.ds(base, bpw)])

    return gather_kernel(item_ids, user_ids, item_embT, user_embT)


def _sc_bias_gather(item_ids, user_ids, ib16, ub16):
    B = item_ids.shape[0]
    info = plsc.get_sparse_core_info()
    NC, NS, L = info.num_cores, info.num_subcores, info.num_lanes
    NW = NC * NS
    bpw = B // NW

    @functools.partial(
        pl.kernel,
        mesh=plsc.VectorSubcoreMesh(**_SC_MESH),
        out_type=(
            jax.ShapeDtypeStruct((B,), jnp.float32),
            jax.ShapeDtypeStruct((B,), jnp.float32),
        ),
        scratch_types=[
            pltpu.VMEM((bpw,), jnp.int32),
            pltpu.VMEM((bpw,), jnp.int32),
            pltpu.VMEM((bpw,), jnp.int32),
            pltpu.VMEM((bpw,), jnp.int32),
            pltpu.VMEM((bpw, L), jnp.float32),
            pltpu.VMEM((bpw, L), jnp.float32),
            pltpu.VMEM((bpw,), jnp.float32),
            pltpu.VMEM((bpw,), jnp.float32),
            pltpu.SemaphoreType.DMA,
        ],
        compiler_params=pltpu.CompilerParams(
            use_tc_tiling_on_sc=False, needs_layout_passes=False),
    )
    def gather_kernel(iid_hbm, uid_hbm, ib_hbm, ub_hbm,
                      bi_out, bu_out,
                      iidx, uidx, iq, uq, ibrow, ubrow, ibv, ubv, sem):
        wid = lax.axis_index("s") * NC + lax.axis_index("c")
        base = wid * bpw
        pltpu.sync_copy(iid_hbm.at[pl.ds(base, bpw)], iidx)
        pltpu.sync_copy(uid_hbm.at[pl.ds(base, bpw)], uidx)
        for j in range(bpw // L):
            s = pl.ds(j * L, L)
            iq[s] = lax.shift_right_logical(iidx[s], 4)
            uq[s] = lax.shift_right_logical(uidx[s], 4)
        c1 = pltpu.async_copy(ib_hbm.at[iq], ibrow, sem)
        c2 = pltpu.async_copy(ub_hbm.at[uq], ubrow, sem)
        c1.wait()
        c2.wait()
        for j in range(bpw // L):
            s = pl.ds(j * L, L)
            rowv = lax.iota(jnp.int32, L) + (j * L)
            ibv[s] = plsc.load_gather(ibrow, [rowv, iidx[s] & 15])
            ubv[s] = plsc.load_gather(ubrow, [rowv, uidx[s] & 15])
        pltpu.sync_copy(ibv, bi_out.at[pl.ds(base, bpw)])
        pltpu.sync_copy(ubv, bu_out.at[pl.ds(base, bpw)])

    return gather_kernel(item_ids, user_ids, ib16, ub16)


def _tc_transpose(U):
    B, D = U.shape

    def body(u_ref, out_ref):
        out_ref[...] = u_ref[...].T

    return pl.pallas_call(
        body,
        out_shape=jax.ShapeDtypeStruct((D, B), jnp.float32),
    )(U)


def _tc_matmul(I, UT, bi, bu, gb):
    B, D = I.shape
    BLK = 256

    def body(i_ref, ut_ref, bi_ref, bu_ref, gb_ref, out_ref):
        acc = lax.dot_general(
            i_ref[...], ut_ref[...], (((1,), (0,)), ((), ())),
            preferred_element_type=jnp.float32)
        out_ref[...] = acc + bi_ref[...] + bu_ref[...] + gb_ref[0]

    return pl.pallas_call(
        body,
        grid=(B // BLK,),
        in_specs=[
            pl.BlockSpec((BLK, D), lambda i: (i, 0)),
            pl.BlockSpec((D, B), lambda i: (0, 0)),
            pl.BlockSpec((BLK, 1), lambda i: (i, 0)),
            pl.BlockSpec((1, B), lambda i: (0, 0)),
            pl.BlockSpec(memory_space=pltpu.SMEM),
        ],
        out_specs=pl.BlockSpec((BLK, B), lambda i: (i, 0)),
        out_shape=jax.ShapeDtypeStruct((B, B), jnp.float32),
    )(I, UT, bi, bu, gb)


def kernel(item_ids, user_ids, item_emb, user_emb, item_bias, user_bias,
           global_bias):
    B = item_ids.shape[0]
    V, D = item_emb.shape
    L = 16
    item_ids = item_ids.astype(jnp.int32)
    user_ids = user_ids.astype(jnp.int32)
    I, U = _sc_emb_gather(item_ids, user_ids, item_emb.T, user_emb.T)
    bi, bu = _sc_bias_gather(item_ids, user_ids,
                             item_bias.reshape(V // L, L),
                             user_bias.reshape(V // L, L))
    UT = _tc_transpose(U)
    gb = jnp.reshape(global_bias.astype(jnp.float32), (1,))
    return _tc_matmul(I, UT, bi.reshape(B, 1), bu.reshape(1, B), gb)


# speedup vs baseline: 1.0078x; 1.0078x over previous
import functools

import jax
import jax.numpy as jnp
from jax import lax
from jax.experimental import pallas as pl
from jax.experimental.pallas import tpu as pltpu
from jax.experimental.pallas import tpu_sc as plsc

_SC_MESH = dict(core_axis_name="c", subcore_axis_name="s")


def _sc_emb_gather(item_ids, user_ids, item_embT, user_embT):
    B = item_ids.shape[0]
    D, V = item_embT.shape
    info = plsc.get_sparse_core_info()
    NC, NS, L = info.num_cores, info.num_subcores, info.num_lanes
    NW = NC * NS
    bpw = B // NW
    RING = 4

    @functools.partial(
        pl.kernel,
        mesh=plsc.VectorSubcoreMesh(**_SC_MESH),
        out_type=(
            jax.ShapeDtypeStruct((B, D), jnp.float32),
            jax.ShapeDtypeStruct((B, D), jnp.float32),
        ),
        scratch_types=[
            pltpu.SMEM((bpw,), jnp.int32),
            pltpu.SMEM((bpw,), jnp.int32),
            pltpu.VMEM((bpw,), jnp.int32),
            pltpu.VMEM((bpw,), jnp.int32),
            pltpu.VMEM((RING, D, 128), jnp.float32),
            pltpu.VMEM((RING, D, 128), jnp.float32),
            pltpu.VMEM((bpw, D), jnp.float32),
            pltpu.VMEM((bpw, D), jnp.float32),
            [pltpu.SemaphoreType.DMA] * RING,
            [pltpu.SemaphoreType.DMA] * RING,
        ],
        compiler_params=pltpu.CompilerParams(
            use_tc_tiling_on_sc=True, needs_layout_passes=False),
    )
    def gather_kernel(iid_hbm, uid_hbm, iembT_hbm, uembT_hbm,
                      i_out, u_out,
                      iids_sm, uids_sm, iidx, uidx, islab, uslab,
                      irows, urows, isem, usem):
        wid = lax.axis_index("s") * NC + lax.axis_index("c")
        base = wid * bpw
        pltpu.sync_copy(iid_hbm.at[pl.ds(base, bpw)], iidx)
        pltpu.sync_copy(uid_hbm.at[pl.ds(base, bpw)], uidx)
        for j in range(bpw // L):
            iv = iidx[pl.ds(j * L, L)]
            uv = uidx[pl.ds(j * L, L)]
            for k in range(L):
                iids_sm[j * L + k] = iv[k]
                uids_sm[j * L + k] = uv[k]

        def issue(r, b):
            ic = pl.multiple_of(iids_sm[r] & ~127, 128)
            uc = pl.multiple_of(uids_sm[r] & ~127, 128)
            pltpu.async_copy(iembT_hbm.at[:, pl.ds(ic, 128)],
                             islab.at[b], isem[b])
            pltpu.async_copy(uembT_hbm.at[:, pl.ds(uc, 128)],
                             uslab.at[b], usem[b])

        for b in range(RING):
            issue(b, b)

        nG = bpw // RING
        rowv = lax.iota(jnp.int32, L)

        def body(g, _):
            for b in range(RING):
                r = g * RING + b
                pltpu.make_async_copy(
                    iembT_hbm.at[:, pl.ds(0, 128)], islab.at[b],
                    isem[b]).wait()
                pltpu.make_async_copy(
                    uembT_hbm.at[:, pl.ds(0, 128)], uslab.at[b],
                    usem[b]).wait()
                iid = iids_sm[r]
                uid = uids_sm[r]
                icol = jnp.broadcast_to(iid & 127, (L,))
                ucol = jnp.broadcast_to(uid & 127, (L,))
                for h in range(D // L):
                    irows[r, pl.ds(h * L, L)] = plsc.load_gather(
                        islab.at[b], [rowv + h * L, icol])
                    urows[r, pl.ds(h * L, L)] = plsc.load_gather(
                        uslab.at[b], [rowv + h * L, ucol])

                @pl.when(g < nG - 1)
                def _():
                    issue(r + RING, b)

            return 0

        lax.fori_loop(0, nG, body, 0)
        pltpu.sync_copy(irows, i_out.at[pl.ds(base, bpw)])
        pltpu.sync_copy(urows, u_out.at[pl.ds(base, bpw)])

    return gather_kernel(item_ids, user_ids, item_embT, user_embT)


def _sc_bias_gather(item_ids, user_ids, ib16, ub16):
    B = item_ids.shape[0]
    info = plsc.get_sparse_core_info()
    NC, NS, L = info.num_cores, info.num_subcores, info.num_lanes
    NW = NC * NS
    bpw = B // NW

    @functools.partial(
        pl.kernel,
        mesh=plsc.VectorSubcoreMesh(**_SC_MESH),
        out_type=(
            jax.ShapeDtypeStruct((B,), jnp.float32),
            jax.ShapeDtypeStruct((B,), jnp.float32),
        ),
        scratch_types=[
            pltpu.VMEM((bpw,), jnp.int32),
            pltpu.VMEM((bpw,), jnp.int32),
            pltpu.VMEM((bpw,), jnp.int32),
            pltpu.VMEM((bpw,), jnp.int32),
            pltpu.VMEM((bpw, L), jnp.float32),
            pltpu.VMEM((bpw, L), jnp.float32),
            pltpu.VMEM((bpw,), jnp.float32),
            pltpu.VMEM((bpw,), jnp.float32),
            pltpu.SemaphoreType.DMA,
        ],
        compiler_params=pltpu.CompilerParams(
            use_tc_tiling_on_sc=False, needs_layout_passes=False),
    )
    def gather_kernel(iid_hbm, uid_hbm, ib_hbm, ub_hbm,
                      bi_out, bu_out,
                      iidx, uidx, iq, uq, ibrow, ubrow, ibv, ubv, sem):
        wid = lax.axis_index("s") * NC + lax.axis_index("c")
        base = wid * bpw
        pltpu.sync_copy(iid_hbm.at[pl.ds(base, bpw)], iidx)
        pltpu.sync_copy(uid_hbm.at[pl.ds(base, bpw)], uidx)
        for j in range(bpw // L):
            s = pl.ds(j * L, L)
            iq[s] = lax.shift_right_logical(iidx[s], 4)
            uq[s] = lax.shift_right_logical(uidx[s], 4)
        c1 = pltpu.async_copy(ib_hbm.at[iq], ibrow, sem)
        c2 = pltpu.async_copy(ub_hbm.at[uq], ubrow, sem)
        c1.wait()
        c2.wait()
        for j in range(bpw // L):
            s = pl.ds(j * L, L)
            rowv = lax.iota(jnp.int32, L) + (j * L)
            ibv[s] = plsc.load_gather(ibrow, [rowv, iidx[s] & 15])
            ubv[s] = plsc.load_gather(ubrow, [rowv, uidx[s] & 15])
        pltpu.sync_copy(ibv, bi_out.at[pl.ds(base, bpw)])
        pltpu.sync_copy(ubv, bu_out.at[pl.ds(base, bpw)])

    return gather_kernel(item_ids, user_ids, ib16, ub16)


def _tc_transpose(U):
    B, D = U.shape

    def body(u_ref, out_ref):
        out_ref[...] = u_ref[...].T

    return pl.pallas_call(
        body,
        out_shape=jax.ShapeDtypeStruct((D, B), jnp.float32),
    )(U)


def _tc_matmul(I, UT, bi, bu, gb):
    B, D = I.shape
    BLK = 1024

    def body(i_ref, ut_ref, bi_ref, bu_ref, gb_ref, out_ref):
        acc = lax.dot_general(
            i_ref[...], ut_ref[...], (((1,), (0,)), ((), ())),
            preferred_element_type=jnp.float32)
        out_ref[...] = acc + bi_ref[...] + bu_ref[...] + gb_ref[0]

    return pl.pallas_call(
        body,
        grid=(B // BLK,),
        in_specs=[
            pl.BlockSpec((BLK, D), lambda i: (i, 0)),
            pl.BlockSpec((D, B), lambda i: (0, 0)),
            pl.BlockSpec((BLK, 1), lambda i: (i, 0)),
            pl.BlockSpec((1, B), lambda i: (0, 0)),
            pl.BlockSpec(memory_space=pltpu.SMEM),
        ],
        out_specs=pl.BlockSpec((BLK, B), lambda i: (i, 0)),
        out_shape=jax.ShapeDtypeStruct((B, B), jnp.float32),
    )(I, UT, bi, bu, gb)


def kernel(item_ids, user_ids, item_emb, user_emb, item_bias, user_bias,
           global_bias):
    B = item_ids.shape[0]
    V, D = item_emb.shape
    L = 16
    item_ids = item_ids.astype(jnp.int32)
    user_ids = user_ids.astype(jnp.int32)
    I, U = _sc_emb_gather(item_ids, user_ids, item_emb.T, user_emb.T)
    bi, bu = _sc_bias_gather(item_ids, user_ids,
                             item_bias.reshape(V // L, L),
                             user_bias.reshape(V // L, L))
    UT = _tc_transpose(U)
    gb = jnp.reshape(global_bias.astype(jnp.float32), (1,))
    return _tc_matmul(I, UT, bi.reshape(B, 1), bu.reshape(1, B), gb)


# speedup vs baseline: 1.0162x; 1.0083x over previous
import functools

import jax
import jax.numpy as jnp
from jax import lax
from jax.experimental import pallas as pl
from jax.experimental.pallas import tpu as pltpu
from jax.experimental.pallas import tpu_sc as plsc

_SC_MESH = dict(core_axis_name="c", subcore_axis_name="s")


def _sc_emb_gather(item_ids, user_ids, item_embT, user_embT):
    B = item_ids.shape[0]
    D, V = item_embT.shape
    info = plsc.get_sparse_core_info()
    NC, NS, L = info.num_cores, info.num_subcores, info.num_lanes
    NW = NC * NS
    bpw = B // NW
    RING = 4

    @functools.partial(
        pl.kernel,
        mesh=plsc.VectorSubcoreMesh(**_SC_MESH),
        out_type=(
            jax.ShapeDtypeStruct((B, D), jnp.float32),
            jax.ShapeDtypeStruct((B, D), jnp.float32),
        ),
        scratch_types=[
            pltpu.SMEM((bpw,), jnp.int32),
            pltpu.SMEM((bpw,), jnp.int32),
            pltpu.VMEM((bpw,), jnp.int32),
            pltpu.VMEM((bpw,), jnp.int32),
            pltpu.VMEM((RING, D, 128), jnp.float32),
            pltpu.VMEM((RING, D, 128), jnp.float32),
            pltpu.VMEM((bpw, D), jnp.float32),
            pltpu.VMEM((bpw, D), jnp.float32),
            [pltpu.SemaphoreType.DMA] * RING,
            [pltpu.SemaphoreType.DMA] * RING,
        ],
        compiler_params=pltpu.CompilerParams(
            use_tc_tiling_on_sc=True, needs_layout_passes=False),
    )
    def gather_kernel(iid_hbm, uid_hbm, iembT_hbm, uembT_hbm,
                      i_out, u_out,
                      iids_sm, uids_sm, iidx, uidx, islab, uslab,
                      irows, urows, isem, usem):
        wid = lax.axis_index("s") * NC + lax.axis_index("c")
        base = wid * bpw
        pltpu.sync_copy(iid_hbm.at[pl.ds(base, bpw)], iidx)
        pltpu.sync_copy(uid_hbm.at[pl.ds(base, bpw)], uidx)
        for j in range(bpw // L):
            iv = iidx[pl.ds(j * L, L)]
            uv = uidx[pl.ds(j * L, L)]
            for k in range(L):
                iids_sm[j * L + k] = iv[k]
                uids_sm[j * L + k] = uv[k]

        def issue(r, b):
            ic = pl.multiple_of(iids_sm[r] & ~127, 128)
            uc = pl.multiple_of(uids_sm[r] & ~127, 128)
            pltpu.async_copy(iembT_hbm.at[:, pl.ds(ic, 128)],
                             islab.at[b], isem[b])
            pltpu.async_copy(uembT_hbm.at[:, pl.ds(uc, 128)],
                             uslab.at[b], usem[b])

        for b in range(RING):
            issue(b, b)

        nG = bpw // RING
        rowv = lax.iota(jnp.int32, L)

        def body(g, _):
            for b in range(RING):
                r = g * RING + b
                pltpu.make_async_copy(
                    iembT_hbm.at[:, pl.ds(0, 128)], islab.at[b],
                    isem[b]).wait()
                pltpu.make_async_copy(
                    uembT_hbm.at[:, pl.ds(0, 128)], uslab.at[b],
                    usem[b]).wait()
                iid = iids_sm[r]
                uid = uids_sm[r]
                icol = jnp.broadcast_to(iid & 127, (L,))
                ucol = jnp.broadcast_to(uid & 127, (L,))
                for h in range(D // L):
                    irows[r, pl.ds(h * L, L)] = plsc.load_gather(
                        islab.at[b], [rowv + h * L, icol])
                    urows[r, pl.ds(h * L, L)] = plsc.load_gather(
                        uslab.at[b], [rowv + h * L, ucol])

                @pl.when(g < nG - 1)
                def _():
                    issue(r + RING, b)

            return 0

        lax.fori_loop(0, nG, body, 0)
        pltpu.sync_copy(irows, i_out.at[pl.ds(base, bpw)])
        pltpu.sync_copy(urows, u_out.at[pl.ds(base, bpw)])

    return gather_kernel(item_ids, user_ids, item_embT, user_embT)


def _sc_bias_gather(item_ids, user_ids, ib16, ub16):
    B = item_ids.shape[0]
    info = plsc.get_sparse_core_info()
    NC, NS, L = info.num_cores, info.num_subcores, info.num_lanes
    NW = NC * NS
    bpw = B // NW

    @functools.partial(
        pl.kernel,
        mesh=plsc.VectorSubcoreMesh(**_SC_MESH),
        out_type=(
            jax.ShapeDtypeStruct((B,), jnp.float32),
            jax.ShapeDtypeStruct((B,), jnp.float32),
        ),
        scratch_types=[
            pltpu.VMEM((bpw,), jnp.int32),
            pltpu.VMEM((bpw,), jnp.int32),
            pltpu.VMEM((bpw,), jnp.int32),
            pltpu.VMEM((bpw,), jnp.int32),
            pltpu.VMEM((bpw, L), jnp.float32),
            pltpu.VMEM((bpw, L), jnp.float32),
            pltpu.VMEM((bpw,), jnp.float32),
            pltpu.VMEM((bpw,), jnp.float32),
            pltpu.SemaphoreType.DMA,
        ],
        compiler_params=pltpu.CompilerParams(
            use_tc_tiling_on_sc=False, needs_layout_passes=False),
    )
    def gather_kernel(iid_hbm, uid_hbm, ib_hbm, ub_hbm,
                      bi_out, bu_out,
                      iidx, uidx, iq, uq, ibrow, ubrow, ibv, ubv, sem):
        wid = lax.axis_index("s") * NC + lax.axis_index("c")
        base = wid * bpw
        pltpu.sync_copy(iid_hbm.at[pl.ds(base, bpw)], iidx)
        pltpu.sync_copy(uid_hbm.at[pl.ds(base, bpw)], uidx)
        for j in range(bpw // L):
            s = pl.ds(j * L, L)
            iq[s] = lax.shift_right_logical(iidx[s], 4)
            uq[s] = lax.shift_right_logical(uidx[s], 4)
        c1 = pltpu.async_copy(ib_hbm.at[iq], ibrow, sem)
        c2 = pltpu.async_copy(ub_hbm.at[uq], ubrow, sem)
        c1.wait()
        c2.wait()
        for j in range(bpw // L):
            s = pl.ds(j * L, L)
            rowv = lax.iota(jnp.int32, L) + (j * L)
            ibv[s] = plsc.load_gather(ibrow, [rowv, iidx[s] & 15])
            ubv[s] = plsc.load_gather(ubrow, [rowv, uidx[s] & 15])
        pltpu.sync_copy(ibv, bi_out.at[pl.ds(base, bpw)])
        pltpu.sync_copy(ubv, bu_out.at[pl.ds(base, bpw)])

    return gather_kernel(item_ids, user_ids, ib16, ub16)


def _tc_transpose(U):
    B, D = U.shape

    def body(u_ref, out_ref):
        out_ref[...] = u_ref[...].T

    return pl.pallas_call(
        body,
        out_shape=jax.ShapeDtypeStruct((D, B), jnp.float32),
    )(U)


def _tc_matmul(I, UT, bi, bu, gb):
    B, D = I.shape
    BLK = 512

    def body(i_ref, ut_ref, bi_ref, bu_ref, gb_ref, out_ref):
        acc = lax.dot_general(
            i_ref[...], ut_ref[...], (((1,), (0,)), ((), ())),
            preferred_element_type=jnp.float32)
        out_ref[...] = acc + bi_ref[...] + bu_ref[...] + gb_ref[0]

    return pl.pallas_call(
        body,
        grid=(B // BLK,),
        in_specs=[
            pl.BlockSpec((BLK, D), lambda i: (i, 0)),
            pl.BlockSpec((D, B), lambda i: (0, 0)),
            pl.BlockSpec((BLK, 1), lambda i: (i, 0)),
            pl.BlockSpec((1, B), lambda i: (0, 0)),
            pl.BlockSpec(memory_space=pltpu.SMEM),
        ],
        out_specs=pl.BlockSpec((BLK, B), lambda i: (i, 0)),
        out_shape=jax.ShapeDtypeStruct((B, B), jnp.float32),
    )(I, UT, bi, bu, gb)


def kernel(item_ids, user_ids, item_emb, user_emb, item_bias, user_bias,
           global_bias):
    B = item_ids.shape[0]
    V, D = item_emb.shape
    L = 16
    item_ids = item_ids.astype(jnp.int32)
    user_ids = user_ids.astype(jnp.int32)
    I, U = _sc_emb_gather(item_ids, user_ids, item_emb.T, user_emb.T)
    bi, bu = _sc_bias_gather(item_ids, user_ids,
                             item_bias.reshape(V // L, L),
                             user_bias.reshape(V // L, L))
    UT = _tc_transpose(U)
    gb = jnp.reshape(global_bias.astype(jnp.float32), (1,))
    return _tc_matmul(I, UT, bi.reshape(B, 1), bu.reshape(1, B), gb)


# speedup vs baseline: 1.4053x; 1.3830x over previous
import functools

import jax
import jax.numpy as jnp
from jax import lax
from jax.experimental import pallas as pl
from jax.experimental.pallas import tpu as pltpu
from jax.experimental.pallas import tpu_sc as plsc

_SC_MESH = dict(core_axis_name="c", subcore_axis_name="s")


def _sc_emb_gather(item_ids, user_ids, item_embT, user_embT):
    B = item_ids.shape[0]
    D, V = item_embT.shape
    info = plsc.get_sparse_core_info()
    NC, NS, L = info.num_cores, info.num_subcores, info.num_lanes
    NW = NC * NS
    bpw = B // NW
    RING = 4

    @functools.partial(
        pl.kernel,
        mesh=plsc.VectorSubcoreMesh(**_SC_MESH),
        out_type=(
            jax.ShapeDtypeStruct((B, D), jnp.float32),
            jax.ShapeDtypeStruct((B, D), jnp.float32),
        ),
        scratch_types=[
            pltpu.SMEM((bpw,), jnp.int32),
            pltpu.SMEM((bpw,), jnp.int32),
            pltpu.VMEM((bpw,), jnp.int32),
            pltpu.VMEM((bpw,), jnp.int32),
            pltpu.VMEM((RING, D, 128), jnp.float32),
            pltpu.VMEM((RING, D, 128), jnp.float32),
            pltpu.VMEM((bpw, D), jnp.float32),
            pltpu.VMEM((bpw, D), jnp.float32),
            [pltpu.SemaphoreType.DMA] * RING,
            [pltpu.SemaphoreType.DMA] * RING,
        ],
        compiler_params=pltpu.CompilerParams(
            use_tc_tiling_on_sc=True, needs_layout_passes=False),
    )
    def gather_kernel(iid_hbm, uid_hbm, iembT_hbm, uembT_hbm,
                      i_out, u_out,
                      iids_sm, uids_sm, iidx, uidx, islab, uslab,
                      irows, urows, isem, usem):
        wid = lax.axis_index("s") * NC + lax.axis_index("c")
        base = wid * bpw
        pltpu.sync_copy(iid_hbm.at[pl.ds(base, bpw)], iidx)
        pltpu.sync_copy(uid_hbm.at[pl.ds(base, bpw)], uidx)
        for j in range(bpw // L):
            iv = iidx[pl.ds(j * L, L)]
            uv = uidx[pl.ds(j * L, L)]
            for k in range(L):
                iids_sm[j * L + k] = iv[k]
                uids_sm[j * L + k] = uv[k]

        def issue(r, b):
            ic = pl.multiple_of(iids_sm[r] & ~127, 128)
            uc = pl.multiple_of(uids_sm[r] & ~127, 128)
            pltpu.async_copy(iembT_hbm.at[:, pl.ds(ic, 128)],
                             islab.at[b], isem[b])
            pltpu.async_copy(uembT_hbm.at[:, pl.ds(uc, 128)],
                             uslab.at[b], usem[b])

        for b in range(RING):
            issue(b, b)

        nG = bpw // RING
        rowv = lax.iota(jnp.int32, L)

        def body(g, _):
            for b in range(RING):
                r = g * RING + b
                pltpu.make_async_copy(
                    iembT_hbm.at[:, pl.ds(0, 128)], islab.at[b],
                    isem[b]).wait()
                pltpu.make_async_copy(
                    uembT_hbm.at[:, pl.ds(0, 128)], uslab.at[b],
                    usem[b]).wait()
                iid = iids_sm[r]
                uid = uids_sm[r]
                icol = jnp.broadcast_to(iid & 127, (L,))
                ucol = jnp.broadcast_to(uid & 127, (L,))
                for h in range(D // L):
                    irows[r, pl.ds(h * L, L)] = plsc.load_gather(
                        islab.at[b], [rowv + h * L, icol])
                    urows[r, pl.ds(h * L, L)] = plsc.load_gather(
                        uslab.at[b], [rowv + h * L, ucol])

                @pl.when(g < nG - 1)
                def _():
                    issue(r + RING, b)

            return 0

        lax.fori_loop(0, nG, body, 0)
        pltpu.sync_copy(irows, i_out.at[pl.ds(base, bpw)])
        pltpu.sync_copy(urows, u_out.at[pl.ds(base, bpw)])

    return gather_kernel(item_ids, user_ids, item_embT, user_embT)


def _sc_bias_gather(item_ids, user_ids, ib16, ub16):
    B = item_ids.shape[0]
    info = plsc.get_sparse_core_info()
    NC, NS, L = info.num_cores, info.num_subcores, info.num_lanes
    NW = NC * NS
    bpw = B // NW

    @functools.partial(
        pl.kernel,
        mesh=plsc.VectorSubcoreMesh(**_SC_MESH),
        out_type=(
            jax.ShapeDtypeStruct((B,), jnp.float32),
            jax.ShapeDtypeStruct((B,), jnp.float32),
        ),
        scratch_types=[
            pltpu.VMEM((bpw,), jnp.int32),
            pltpu.VMEM((bpw,), jnp.int32),
            pltpu.VMEM((bpw,), jnp.int32),
            pltpu.VMEM((bpw,), jnp.int32),
            pltpu.VMEM((bpw, L), jnp.float32),
            pltpu.VMEM((bpw, L), jnp.float32),
            pltpu.VMEM((bpw,), jnp.float32),
            pltpu.VMEM((bpw,), jnp.float32),
            pltpu.SemaphoreType.DMA,
        ],
        compiler_params=pltpu.CompilerParams(
            use_tc_tiling_on_sc=False, needs_layout_passes=False),
    )
    def gather_kernel(iid_hbm, uid_hbm, ib_hbm, ub_hbm,
                      bi_out, bu_out,
                      iidx, uidx, iq, uq, ibrow, ubrow, ibv, ubv, sem):
        wid = lax.axis_index("s") * NC + lax.axis_index("c")
        base = wid * bpw
        pltpu.sync_copy(iid_hbm.at[pl.ds(base, bpw)], iidx)
        pltpu.sync_copy(uid_hbm.at[pl.ds(base, bpw)], uidx)
        for j in range(bpw // L):
            s = pl.ds(j * L, L)
            iq[s] = lax.shift_right_logical(iidx[s], 4)
            uq[s] = lax.shift_right_logical(uidx[s], 4)
        c1 = pltpu.async_copy(ib_hbm.at[iq], ibrow, sem)
        c2 = pltpu.async_copy(ub_hbm.at[uq], ubrow, sem)
        c1.wait()
        c2.wait()
        for j in range(bpw // L):
            s = pl.ds(j * L, L)
            rowv = lax.iota(jnp.int32, L) + (j * L)
            ibv[s] = plsc.load_gather(ibrow, [rowv, iidx[s] & 15])
            ubv[s] = plsc.load_gather(ubrow, [rowv, uidx[s] & 15])
        pltpu.sync_copy(ibv, bi_out.at[pl.ds(base, bpw)])
        pltpu.sync_copy(ubv, bu_out.at[pl.ds(base, bpw)])

    return gather_kernel(item_ids, user_ids, ib16, ub16)


def _tc_transpose(U):
    B, D = U.shape

    def body(u_ref, out_ref):
        out_ref[...] = u_ref[...].T

    return pl.pallas_call(
        body,
        out_shape=jax.ShapeDtypeStruct((D, B), jnp.float32),
    )(U)


def _tc_matmul(I2, UT2):
    B, K = I2.shape
    BLK = 512

    def body(i_ref, ut_ref, out_ref):
        out_ref[...] = lax.dot_general(
            i_ref[...], ut_ref[...], (((1,), (0,)), ((), ())),
            preferred_element_type=jnp.float32)

    return pl.pallas_call(
        body,
        grid=(B // BLK,),
        in_specs=[
            pl.BlockSpec((BLK, K), lambda i: (i, 0)),
            pl.BlockSpec((K, B), lambda i: (0, 0)),
        ],
        out_specs=pl.BlockSpec((BLK, B), lambda i: (i, 0)),
        out_shape=jax.ShapeDtypeStruct((B, B), jnp.float32),
    )(I2, UT2)


def kernel(item_ids, user_ids, item_emb, user_emb, item_bias, user_bias,
           global_bias):
    B = item_ids.shape[0]
    V, D = item_emb.shape
    L = 16
    item_ids = item_ids.astype(jnp.int32)
    user_ids = user_ids.astype(jnp.int32)
    I, U = _sc_emb_gather(item_ids, user_ids, item_emb.T, user_emb.T)
    bi, bu = _sc_bias_gather(item_ids, user_ids,
                             item_bias.reshape(V // L, L),
                             user_bias.reshape(V // L, L))
    UT = _tc_transpose(U)
    gb = global_bias.astype(jnp.float32)
    ones = jnp.ones((B, 1), jnp.float32)
    zeros = jnp.zeros((B, 6), jnp.float32)
    I2 = jnp.concatenate([I, bi.reshape(B, 1), ones, zeros], axis=1)
    UT2 = jnp.concatenate(
        [UT, ones.reshape(1, B), (bu + gb).reshape(1, B),
         zeros.reshape(6, B)], axis=0)
    return _tc_matmul(I2, UT2)
